# baseline (speedup 1.0000x reference)
import jax
import jax.numpy as jnp
from jax.experimental import pallas as pl
from jax.experimental.pallas import tpu as pltpu

S = 32768
A = 1024
N = 256
BLOCK = 2048
GRID = S // BLOCK


def _body(x_ref, d_ref, o_ref):
    i = pl.program_id(0)

    @pl.when(i < GRID - 1)
    def _copy():
        o_ref[...] = x_ref[...]

    @pl.when(i == GRID - 1)
    def _fixup():
        o_ref[0:BLOCK - N, :] = x_ref[0:BLOCK - N, :]
        o_ref[BLOCK - N:BLOCK, :] = (
            x_ref[BLOCK - 2 * N:BLOCK - N, :] + d_ref[...]
        )


def kernel(expected, drift):
    drift2d = drift.reshape(N, 1)
    return pl.pallas_call(
        _body,
        grid=(GRID,),
        in_specs=[
            pl.BlockSpec((BLOCK, A), lambda i: (i, 0)),
            pl.BlockSpec((N, 1), lambda i: (0, 0)),
        ],
        out_specs=pl.BlockSpec((BLOCK, A), lambda i: (i, 0)),
        out_shape=jax.ShapeDtypeStruct((S, A), expected.dtype),
        compiler_params=pltpu.CompilerParams(
            dimension_semantics=("parallel",),
        ),
    )(expected, drift2d)


# speedup vs baseline: 1.0003x; 1.0003x over previous
import jax
import jax.numpy as jnp
from jax.experimental import pallas as pl
from jax.experimental.pallas import tpu as pltpu

S = 32768
A = 1024
N = 256
B = 2048
M = 6
LAG = 2
NCH = S // B


def _body(x_ref, d_ref, o_ref, *rest):
    bufs = rest[:M]
    isem, osem = rest[M], rest[M + 1]
    cins = [None] * NCH
    couts = [None] * NCH
    waited = set()

    def start_in(i):
        b = i % M
        c = pltpu.make_async_copy(
            x_ref.at[pl.ds(i * B, B), :], bufs[b], isem.at[b])
        c.start()
        cins[i] = c

    for i in range(M):
        start_in(i)
    for i in range(NCH):
        b = i % M
        cins[i].wait()
        if i == NCH - 1:
            bufs[b][B - N:B, :] = bufs[b][B - 2 * N:B - N, :] + d_ref[...]
        c = pltpu.make_async_copy(
            bufs[b], o_ref.at[pl.ds(i * B, B), :], osem.at[b])
        c.start()
        couts[i] = c
        j = i - LAG
        if j >= 0 and j + M < NCH:
            couts[j].wait()
            waited.add(j)
            start_in(j + M)
    for i in range(NCH):
        if i not in waited:
            couts[i].wait()


def kernel(expected, drift):
    drift2d = drift.reshape(N, 1)
    return pl.pallas_call(
        _body,
        in_specs=[
            pl.BlockSpec(memory_space=pltpu.MemorySpace.HBM),
            pl.BlockSpec(memory_space=pltpu.MemorySpace.VMEM),
        ],
        out_specs=pl.BlockSpec(memory_space=pltpu.MemorySpace.HBM),
        out_shape=jax.ShapeDtypeStruct((S, A), expected.dtype),
        scratch_shapes=(
            [pltpu.VMEM((B, A), jnp.float32) for _ in range(M)]
            + [pltpu.SemaphoreType.DMA((M,)), pltpu.SemaphoreType.DMA((M,))]
        ),
    )(expected, drift2d)
